# baseline (speedup 1.0000x reference)
import functools

import jax
import jax.numpy as jnp
from jax import lax
from jax.experimental import pallas as pl
from jax.experimental.pallas import tpu as pltpu
from jax.experimental.pallas import tpu_sc as plsc

B = 16384
EMB = 32
H1 = 64
NF = 26
NU = 1_000_000
NG = 100_000

_NC, _NS = 2, 16
_NW = _NC * _NS
_BPW = B // _NW
_L = 16
_HP = EMB // 2

_TCU = (NU + 127) // 128
_TCG = (NG + 127) // 128
_CST = 1024


def _pack_body(in_ref, out_ref):
    for c in range(_CST):
        sl = slice(128 * c, 128 * c + 128)
        for h in range(2):
            a = lax.bitcast_convert_type(in_ref[8 * h:8 * h + 8, sl],
                                         jnp.int32)
            b = lax.bitcast_convert_type(in_ref[16 + 8 * h:24 + 8 * h, sl],
                                         jnp.int32)
            out_ref[16 * c + 8 * h:16 * c + 8 * h + 8, :] = (
                lax.shift_right_logical(a, 16) | (b & ~0xFFFF))


def _detile_pack(tabT, tcols):
    steps = (tcols + _CST - 1) // _CST
    return pl.pallas_call(
        _pack_body,
        grid=(steps,),
        in_specs=[pl.BlockSpec((EMB, _CST * 128), lambda j: (0, j))],
        out_specs=pl.BlockSpec((_CST * _HP, 128), lambda j: (j, 0)),
        out_shape=jax.ShapeDtypeStruct((_HP * tcols, 128), jnp.int32),
    )(tabT)


def _sc_gather_t(u, g, put, pgt):
    mesh = plsc.VectorSubcoreMesh(core_axis_name="c", subcore_axis_name="s")

    @functools.partial(
        pl.kernel,
        mesh=mesh,
        out_type=[
            jax.ShapeDtypeStruct((_HP, B), jnp.int32),
            jax.ShapeDtypeStruct((_HP, B), jnp.int32),
        ],
        scratch_types=[
            pltpu.VMEM((_BPW,), jnp.int32),
            pltpu.VMEM((_BPW,), jnp.int32),
            pltpu.VMEM((_HP, _BPW), jnp.int32),
            pltpu.VMEM((_HP, _BPW), jnp.int32),
            pltpu.VMEM((_HP, _BPW), jnp.int32),
            pltpu.VMEM((_HP, _BPW), jnp.int32),
            pltpu.SemaphoreType.DMA,
        ],
        compiler_params=pltpu.CompilerParams(use_tc_tiling_on_sc=False),
    )
    def gather(u_hbm, g_hbm, put_hbm, pgt_hbm, ue_out, ge_out,
               u_v, g_v, iu_v, ig_v, ru_v, rg_v, sem):
        wid = lax.axis_index("s") * _NC + lax.axis_index("c")
        base = wid * _BPW
        pltpu.sync_copy(u_hbm.at[pl.ds(base, _BPW)], u_v)
        pltpu.sync_copy(g_hbm.at[pl.ds(base, _BPW)], g_v)

        def build(c, carry):
            uv = u_v[pl.ds(c * _L, _L)]
            gv = g_v[pl.ds(c * _L, _L)]
            ub = (lax.shift_right_logical(uv, 7) * (_HP * 128)) + (uv & 127)
            gb = (lax.shift_right_logical(gv, 7) * (_HP * 128)) + (gv & 127)
            for q in range(_HP):
                iu_v[q, pl.ds(c * _L, _L)] = ub + q * 128
                ig_v[q, pl.ds(c * _L, _L)] = gb + q * 128
            return carry

        lax.fori_loop(0, _BPW // _L, build, 0)

        copies = []
        for q in range(_HP):
            copies.append(pltpu.async_copy(
                put_hbm.at[iu_v.at[q]], ru_v.at[q], sem))
            copies.append(pltpu.async_copy(
                pgt_hbm.at[ig_v.at[q]], rg_v.at[q], sem))
        for cpy in copies:
            cpy.wait()

        pltpu.sync_copy(ru_v, ue_out.at[:, pl.ds(base, _BPW)])
        pltpu.sync_copy(rg_v, ge_out.at[:, pl.ds(base, _BPW)])

    return gather(u, g, put, pgt)


_N = 4096
_GT = B // _N


def _unpack_lo(x):
    return lax.bitcast_convert_type(x << 16, jnp.float32)


def _unpack_hi(x):
    return lax.bitcast_convert_type(x & ~0xFFFF, jnp.float32)


def _mlp_t_body(up, gp, ft, wf, bfc, w1ua, w1ub, w1ga, w1gb, w1f, b1c,
                w2, b2c, out):
    fe = jnp.dot(wf[...], ft[...], preferred_element_type=jnp.float32) + bfc[...]
    x = up[...]
    y = gp[...]
    h = (jnp.dot(w1ua[...], _unpack_lo(x), preferred_element_type=jnp.float32)
         + jnp.dot(w1ub[...], _unpack_hi(x), preferred_element_type=jnp.float32)
         + jnp.dot(w1ga[...], _unpack_lo(y), preferred_element_type=jnp.float32)
         + jnp.dot(w1gb[...], _unpack_hi(y), preferred_element_type=jnp.float32)
         + jnp.dot(w1f[...], fe, preferred_element_type=jnp.float32)
         + b1c[...])
    h = jnp.maximum(h, 0.0)
    out[...] = jnp.dot(w2[...], h, preferred_element_type=jnp.float32) + b2c[...]


def _tc_mlp_t(up, gp, ft, wf, bfc, w1ua, w1ub, w1ga, w1gb, w1f, b1c,
              w2, b2c, interpret=False):
    col = lambda i: (0, i)
    rep = lambda i: (0, 0)
    return pl.pallas_call(
        _mlp_t_body,
        grid=(_GT,),
        in_specs=[
            pl.BlockSpec((_HP, _N), col),
            pl.BlockSpec((_HP, _N), col),
            pl.BlockSpec((NF, _N), col),
            pl.BlockSpec((EMB, NF), rep),
            pl.BlockSpec((EMB, 1), rep),
            pl.BlockSpec((H1, _HP), rep),
            pl.BlockSpec((H1, _HP), rep),
            pl.BlockSpec((H1, _HP), rep),
            pl.BlockSpec((H1, _HP), rep),
            pl.BlockSpec((H1, EMB), rep),
            pl.BlockSpec((H1, 1), rep),
            pl.BlockSpec((1, H1), rep),
            pl.BlockSpec((1, 1), rep),
        ],
        out_specs=pl.BlockSpec((1, _N), col),
        out_shape=jax.ShapeDtypeStruct((1, B), jnp.float32),
        interpret=interpret,
    )(up, gp, ft, wf, bfc, w1ua, w1ub, w1ga, w1gb, w1f, b1c, w2, b2c)


def kernel(u, g, f, user_emb, game_emb, Wf, bf, W1, b1, W2, b2):
    put = _detile_pack(user_emb.T, _TCU).reshape(-1)
    pgt = _detile_pack(game_emb.T, _TCG).reshape(-1)
    ue_pk, ge_pk = _sc_gather_t(u, g, put, pgt)
    outT = _tc_mlp_t(
        ue_pk, ge_pk, f.T,
        Wf,
        bf.reshape(EMB, 1),
        W1[:, 0:_HP],
        W1[:, _HP:EMB],
        W1[:, EMB:EMB + _HP],
        W1[:, EMB + _HP:2 * EMB],
        W1[:, 2 * EMB:],
        b1.reshape(H1, 1),
        W2,
        b2.reshape(1, 1),
    )
    return outT.reshape(B, 1)


# speedup vs baseline: 1.0197x; 1.0197x over previous
import functools

import jax
import jax.numpy as jnp
from jax import lax
from jax.experimental import pallas as pl
from jax.experimental.pallas import tpu as pltpu
from jax.experimental.pallas import tpu_sc as plsc

B = 16384
EMB = 32
H1 = 64
NF = 26
NU = 1_000_000
NG = 100_000

_NC, _NS = 2, 16
_NW = _NC * _NS
_BPW = B // _NW
_L = 16
_HP = EMB // 2

_TCU = (NU + 127) // 128
_TCG = (NG + 127) // 128
_CST = 512


def _pack_body(in_ref, out_ref):
    for c in range(_CST):
        sl = slice(128 * c, 128 * c + 128)
        for h in range(2):
            a = lax.bitcast_convert_type(in_ref[8 * h:8 * h + 8, sl],
                                         jnp.int32)
            b = lax.bitcast_convert_type(in_ref[16 + 8 * h:24 + 8 * h, sl],
                                         jnp.int32)
            out_ref[16 * c + 8 * h:16 * c + 8 * h + 8, :] = (
                lax.shift_right_logical(a, 16) | (b & ~0xFFFF))


def _detile_pack(tabT, tcols):
    steps = (tcols + _CST - 1) // _CST
    return pl.pallas_call(
        _pack_body,
        grid=(steps,),
        in_specs=[pl.BlockSpec((EMB, _CST * 128), lambda j: (0, j))],
        out_specs=pl.BlockSpec((_CST * _HP, 128), lambda j: (j, 0)),
        out_shape=jax.ShapeDtypeStruct((_HP * tcols, 128), jnp.int32),
    )(tabT)


def _sc_gather_one(idx, pkt):
    mesh = plsc.VectorSubcoreMesh(core_axis_name="c", subcore_axis_name="s")

    @functools.partial(
        pl.kernel,
        mesh=mesh,
        out_type=jax.ShapeDtypeStruct((_HP, B), jnp.int32),
        scratch_types=[
            pltpu.VMEM((_BPW,), jnp.int32),
            pltpu.VMEM((_HP, _BPW), jnp.int32),
            pltpu.VMEM((_HP, _BPW), jnp.int32),
            pltpu.SemaphoreType.DMA,
        ],
        compiler_params=pltpu.CompilerParams(use_tc_tiling_on_sc=False),
    )
    def gather(i_hbm, pkt_hbm, e_out, i_v, ix_v, r_v, sem):
        wid = lax.axis_index("s") * _NC + lax.axis_index("c")
        base = wid * _BPW
        pltpu.sync_copy(i_hbm.at[pl.ds(base, _BPW)], i_v)

        def build(c, carry):
            iv = i_v[pl.ds(c * _L, _L)]
            ib = (lax.shift_right_logical(iv, 7) * (_HP * 128)) + (iv & 127)
            for q in range(_HP):
                ix_v[q, pl.ds(c * _L, _L)] = ib + q * 128
            return carry

        lax.fori_loop(0, _BPW // _L, build, 0)

        copies = []
        for q in range(_HP):
            copies.append(pltpu.async_copy(
                pkt_hbm.at[ix_v.at[q]], r_v.at[q], sem))
        for cpy in copies:
            cpy.wait()

        pltpu.sync_copy(r_v, e_out.at[:, pl.ds(base, _BPW)])

    return gather(idx, pkt)


_N = 4096
_GT = B // _N


def _unpack_lo(x):
    return lax.bitcast_convert_type(x << 16, jnp.float32)


def _unpack_hi(x):
    return lax.bitcast_convert_type(x & ~0xFFFF, jnp.float32)


def _mlp_t_body(up, gp, ft, wf, bfc, w1ua, w1ub, w1ga, w1gb, w1f, b1c,
                w2, b2c, out):
    fe = jnp.dot(wf[...], ft[...], preferred_element_type=jnp.float32) + bfc[...]
    x = up[...]
    y = gp[...]
    h = (jnp.dot(w1ua[...], _unpack_lo(x), preferred_element_type=jnp.float32)
         + jnp.dot(w1ub[...], _unpack_hi(x), preferred_element_type=jnp.float32)
         + jnp.dot(w1ga[...], _unpack_lo(y), preferred_element_type=jnp.float32)
         + jnp.dot(w1gb[...], _unpack_hi(y), preferred_element_type=jnp.float32)
         + jnp.dot(w1f[...], fe, preferred_element_type=jnp.float32)
         + b1c[...])
    h = jnp.maximum(h, 0.0)
    out[...] = jnp.dot(w2[...], h, preferred_element_type=jnp.float32) + b2c[...]


def _tc_mlp_t(up, gp, ft, wf, bfc, w1ua, w1ub, w1ga, w1gb, w1f, b1c,
              w2, b2c, interpret=False):
    col = lambda i: (0, i)
    rep = lambda i: (0, 0)
    return pl.pallas_call(
        _mlp_t_body,
        grid=(_GT,),
        in_specs=[
            pl.BlockSpec((_HP, _N), col),
            pl.BlockSpec((_HP, _N), col),
            pl.BlockSpec((NF, _N), col),
            pl.BlockSpec((EMB, NF), rep),
            pl.BlockSpec((EMB, 1), rep),
            pl.BlockSpec((H1, _HP), rep),
            pl.BlockSpec((H1, _HP), rep),
            pl.BlockSpec((H1, _HP), rep),
            pl.BlockSpec((H1, _HP), rep),
            pl.BlockSpec((H1, EMB), rep),
            pl.BlockSpec((H1, 1), rep),
            pl.BlockSpec((1, H1), rep),
            pl.BlockSpec((1, 1), rep),
        ],
        out_specs=pl.BlockSpec((1, _N), col),
        out_shape=jax.ShapeDtypeStruct((1, B), jnp.float32),
        interpret=interpret,
    )(up, gp, ft, wf, bfc, w1ua, w1ub, w1ga, w1gb, w1f, b1c, w2, b2c)


def kernel(u, g, f, user_emb, game_emb, Wf, bf, W1, b1, W2, b2):
    pgt = _detile_pack(game_emb.T, _TCG).reshape(-1)
    ge_pk = _sc_gather_one(g, pgt)
    put = _detile_pack(user_emb.T, _TCU).reshape(-1)
    ue_pk = _sc_gather_one(u, put)
    outT = _tc_mlp_t(
        ue_pk, ge_pk, f.T,
        Wf,
        bf.reshape(EMB, 1),
        W1[:, 0:_HP],
        W1[:, _HP:EMB],
        W1[:, EMB:EMB + _HP],
        W1[:, EMB + _HP:2 * EMB],
        W1[:, 2 * EMB:],
        b1.reshape(H1, 1),
        W2,
        b2.reshape(1, 1),
    )
    return outT.reshape(B, 1)
